# baseline (speedup 1.0000x reference)
import functools

import jax
import jax.numpy as jnp
from jax.experimental import pallas as pl
from jax.experimental.pallas import tpu as pltpu


def _moe_body(x_ref, p_ref, w1p_ref, w2_ref, out_ref,
              wg_ref, wu_ref, w2b_ref,
              *, alpha, limit, linear_offset):
    t = pl.program_id(1)

    @pl.when(t == 0)
    def _unpack_weights():
        u = jax.lax.bitcast_convert_type(w1p_ref[0], jnp.uint32)
        gate_w = jax.lax.bitcast_convert_type(u << jnp.uint32(16),
                                              jnp.float32)
        up_w = jax.lax.bitcast_convert_type(u & jnp.uint32(0xFFFF0000),
                                            jnp.float32)
        wg_ref[...] = gate_w.astype(jnp.bfloat16)
        wu_ref[...] = up_w.astype(jnp.bfloat16)
        w2b_ref[...] = w2_ref[0].astype(jnp.bfloat16)

    x = x_ref[0].astype(jnp.bfloat16)
    gate = jax.lax.dot_general(
        x, wg_ref[...], (((1,), (0,)), ((), ())),
        preferred_element_type=jnp.float32)
    up = jax.lax.dot_general(
        x, wu_ref[...], (((1,), (0,)), ((), ())),
        preferred_element_type=jnp.float32)
    gate = jnp.minimum(gate, limit)
    up = jnp.clip(up, -limit, limit)
    glu = gate * jax.nn.sigmoid(alpha * gate)
    inter = glu * (up + linear_offset) * p_ref[0]
    out_ref[0] = jax.lax.dot_general(
        inter.astype(jnp.bfloat16), w2b_ref[...], (((1,), (0,)), ((), ())),
        preferred_element_type=jnp.float32)


def kernel(hidden_states, tokens_per_expert, permuted_probs,
           gate_and_up_projs, down_projs):
    n_experts, dim, two_inter = gate_and_up_projs.shape
    inter = down_projs.shape[1]
    tokens = hidden_states.shape[0]
    tpe = tokens // n_experts

    bt = 256
    x = hidden_states.reshape(n_experts, tpe, dim)
    p = permuted_probs.reshape(n_experts, tpe, 1)
    u = jax.lax.bitcast_convert_type(gate_and_up_projs, jnp.uint32)
    b = (u + jnp.uint32(0x7FFF) + ((u >> jnp.uint32(16)) & jnp.uint32(1))
         ) >> jnp.uint32(16)
    b4 = b.reshape(n_experts, dim, inter, 2)
    w1p = jax.lax.bitcast_convert_type(
        b4[..., 0] | (b4[..., 1] << jnp.uint32(16)), jnp.float32)

    out = pl.pallas_call(
        functools.partial(_moe_body, alpha=1.702, limit=7.0,
                          linear_offset=1.0),
        grid=(n_experts, tpe // bt),
        in_specs=[
            pl.BlockSpec((1, bt, dim), lambda e, t: (e, t, 0)),
            pl.BlockSpec((1, bt, 1), lambda e, t: (e, t, 0)),
            pl.BlockSpec((1, dim, inter), lambda e, t: (e, 0, 0)),
            pl.BlockSpec((1, inter, dim), lambda e, t: (e, 0, 0)),
        ],
        out_specs=pl.BlockSpec((1, bt, dim), lambda e, t: (e, t, 0)),
        out_shape=jax.ShapeDtypeStruct((n_experts, tpe, dim), jnp.float32),
        scratch_shapes=[
            pltpu.VMEM((dim, inter), jnp.bfloat16),
            pltpu.VMEM((dim, inter), jnp.bfloat16),
            pltpu.VMEM((inter, dim), jnp.bfloat16),
        ],
        compiler_params=pltpu.CompilerParams(
            dimension_semantics=("parallel", "arbitrary"),
        ),
    )(x, p, w1p, down_projs)
    return out.reshape(tokens, dim)


# speedup vs baseline: 1.8777x; 1.8777x over previous
import functools

import jax
import jax.numpy as jnp
from jax.experimental import pallas as pl
from jax.experimental.pallas import tpu as pltpu


def _moe_body(x_ref, p_ref, w1_ref, w2_ref, out_ref,
              wg_ref, wu_ref, w2b_ref,
              *, alpha, limit, linear_offset):
    t = pl.program_id(1)
    half_cols = w1_ref.shape[2]
    half_pairs = half_cols // 2

    @pl.when(t < 2)
    def _deinterleave_half():
        w1h = w1_ref[0].astype(jnp.bfloat16)
        row = jax.lax.broadcasted_iota(jnp.int32, (half_cols, half_pairs), 0)
        col = jax.lax.broadcasted_iota(jnp.int32, (half_cols, half_pairs), 1)
        sel_g = (row == 2 * col).astype(jnp.bfloat16)
        sel_u = (row == 2 * col + 1).astype(jnp.bfloat16)
        off = t * half_pairs
        wg_ref[:, pl.ds(off, half_pairs)] = jax.lax.dot_general(
            w1h, sel_g, (((1,), (0,)), ((), ())),
            preferred_element_type=jnp.float32).astype(jnp.bfloat16)
        wu_ref[:, pl.ds(off, half_pairs)] = jax.lax.dot_general(
            w1h, sel_u, (((1,), (0,)), ((), ())),
            preferred_element_type=jnp.float32).astype(jnp.bfloat16)

    @pl.when(t == 0)
    def _cast_w2():
        w2b_ref[...] = w2_ref[0].astype(jnp.bfloat16)

    @pl.when(t > 0)
    def _compute_tile():
        x = x_ref[0].astype(jnp.bfloat16)
        gate = jax.lax.dot_general(
            x, wg_ref[...], (((1,), (0,)), ((), ())),
            preferred_element_type=jnp.float32)
        up = jax.lax.dot_general(
            x, wu_ref[...], (((1,), (0,)), ((), ())),
            preferred_element_type=jnp.float32)
        gate = jnp.minimum(gate, limit)
        up = jnp.clip(up, -limit, limit)
        glu = gate * jax.nn.sigmoid(alpha * gate)
        inter = glu * (up + linear_offset) * p_ref[0]
        out_ref[0] = jax.lax.dot_general(
            inter.astype(jnp.bfloat16), w2b_ref[...], (((1,), (0,)), ((), ())),
            preferred_element_type=jnp.float32)


def kernel(hidden_states, tokens_per_expert, permuted_probs,
           gate_and_up_projs, down_projs):
    n_experts, dim, two_inter = gate_and_up_projs.shape
    inter = down_projs.shape[1]
    tokens = hidden_states.shape[0]
    tpe = tokens // n_experts

    bt = 128
    n_tiles = tpe // bt
    x = hidden_states.reshape(n_experts, tpe, dim)
    p = permuted_probs.reshape(n_experts, tpe, 1)

    def _tile_idx(t):
        return jnp.maximum(t - 1, 0)

    out = pl.pallas_call(
        functools.partial(_moe_body, alpha=1.702, limit=7.0,
                          linear_offset=1.0),
        grid=(n_experts, n_tiles + 1),
        in_specs=[
            pl.BlockSpec((1, bt, dim), lambda e, t: (e, _tile_idx(t), 0)),
            pl.BlockSpec((1, bt, 1), lambda e, t: (e, _tile_idx(t), 0)),
            pl.BlockSpec((1, dim, inter),
                         lambda e, t: (e, 0, jnp.minimum(t, 1))),
            pl.BlockSpec((1, inter, dim), lambda e, t: (e, 0, 0)),
        ],
        out_specs=pl.BlockSpec((1, bt, dim), lambda e, t: (e, _tile_idx(t), 0)),
        out_shape=jax.ShapeDtypeStruct((n_experts, tpe, dim), jnp.float32),
        scratch_shapes=[
            pltpu.VMEM((dim, inter), jnp.bfloat16),
            pltpu.VMEM((dim, inter), jnp.bfloat16),
            pltpu.VMEM((inter, dim), jnp.bfloat16),
        ],
        compiler_params=pltpu.CompilerParams(
            dimension_semantics=("parallel", "arbitrary"),
        ),
    )(x, p, gate_and_up_projs, down_projs)
    return out.reshape(tokens, dim)


# speedup vs baseline: 2.4373x; 1.2980x over previous
import functools

import jax
import jax.numpy as jnp
from jax.experimental import pallas as pl
from jax.experimental.pallas import tpu as pltpu


def _moe_body(x_ref, p_ref, w1_ref, w2_ref, out_ref,
              wg_ref, wu_ref, w2b_ref,
              *, alpha, limit, linear_offset, n_experts):
    e = pl.program_id(0)
    t = pl.program_id(1)
    qcols = w1_ref.shape[2]
    qpairs = qcols // 2
    load_bank = jax.lax.rem(e, 2)
    c = jnp.maximum(e - 1, 0)
    comp_bank = jax.lax.rem(c, 2)

    @pl.when((e < n_experts) & (t < 4))
    def _select_quarter():
        w1q = w1_ref[0].astype(jnp.bfloat16)
        row = jax.lax.broadcasted_iota(jnp.int32, (qcols, qpairs), 0)
        col = jax.lax.broadcasted_iota(jnp.int32, (qcols, qpairs), 1)
        sel_g = (row == 2 * col).astype(jnp.bfloat16)
        sel_u = (row == 2 * col + 1).astype(jnp.bfloat16)
        off = t * qpairs
        wg_ref[load_bank, :, pl.ds(off, qpairs)] = jax.lax.dot_general(
            w1q, sel_g, (((1,), (0,)), ((), ())),
            preferred_element_type=jnp.float32).astype(jnp.bfloat16)
        wu_ref[load_bank, :, pl.ds(off, qpairs)] = jax.lax.dot_general(
            w1q, sel_u, (((1,), (0,)), ((), ())),
            preferred_element_type=jnp.float32).astype(jnp.bfloat16)

    @pl.when((e < n_experts) & ((t == 4) | (t == 5)))
    def _cast_w2_half():
        half_rows = w2_ref.shape[1]
        off = (t - 4) * half_rows
        w2b_ref[load_bank, pl.ds(off, half_rows), :] = (
            w2_ref[0].astype(jnp.bfloat16))

    @pl.when(e >= 1)
    def _compute_tile():
        x = x_ref[0].astype(jnp.bfloat16)
        gate = jax.lax.dot_general(
            x, wg_ref[comp_bank], (((1,), (0,)), ((), ())),
            preferred_element_type=jnp.float32)
        up = jax.lax.dot_general(
            x, wu_ref[comp_bank], (((1,), (0,)), ((), ())),
            preferred_element_type=jnp.float32)
        gate = jnp.minimum(gate, limit)
        up = jnp.clip(up, -limit, limit)
        glu = gate * jax.nn.sigmoid(alpha * gate)
        inter = glu * (up + linear_offset) * p_ref[0]
        out_ref[0] = jax.lax.dot_general(
            inter.astype(jnp.bfloat16), w2b_ref[comp_bank],
            (((1,), (0,)), ((), ())),
            preferred_element_type=jnp.float32)


def kernel(hidden_states, tokens_per_expert, permuted_probs,
           gate_and_up_projs, down_projs):
    n_experts, dim, two_inter = gate_and_up_projs.shape
    inter = down_projs.shape[1]
    tokens = hidden_states.shape[0]
    tpe = tokens // n_experts
    e_last = n_experts - 1

    bt = 256
    n_tiles = tpe // bt
    x = hidden_states.reshape(n_experts, tpe, dim)
    p = permuted_probs.reshape(n_experts, tpe, 1)

    def _xpo_idx(e, t):
        return (jnp.maximum(e - 1, 0), jnp.where(e == 0, 0, t), 0)

    def _w1_idx(e, t):
        q = jnp.where(e >= n_experts, 3, jnp.clip(t, 0, 3))
        return (jnp.minimum(e, e_last), 0, q)

    def _w2_idx(e, t):
        ex = jnp.clip(jnp.where(t < 4, e - 1, e), 0, e_last)
        h = jnp.where((e >= n_experts) | (t < 4), 1, jnp.clip(t - 4, 0, 1))
        return (ex, h, 0)

    out = pl.pallas_call(
        functools.partial(_moe_body, alpha=1.702, limit=7.0,
                          linear_offset=1.0, n_experts=n_experts),
        grid=(n_experts + 1, n_tiles),
        in_specs=[
            pl.BlockSpec((1, bt, dim), _xpo_idx),
            pl.BlockSpec((1, bt, 1), _xpo_idx),
            pl.BlockSpec((1, dim, two_inter // 4), _w1_idx),
            pl.BlockSpec((1, inter // 2, dim), _w2_idx),
        ],
        out_specs=pl.BlockSpec((1, bt, dim), _xpo_idx),
        out_shape=jax.ShapeDtypeStruct((n_experts, tpe, dim), jnp.float32),
        scratch_shapes=[
            pltpu.VMEM((2, dim, inter), jnp.bfloat16),
            pltpu.VMEM((2, dim, inter), jnp.bfloat16),
            pltpu.VMEM((2, inter, dim), jnp.bfloat16),
        ],
        compiler_params=pltpu.CompilerParams(
            dimension_semantics=("arbitrary", "arbitrary"),
        ),
    )(x, p, gate_and_up_projs, down_projs)
    return out.reshape(tokens, dim)


# speedup vs baseline: 2.6541x; 1.0889x over previous
import functools

import jax
import jax.numpy as jnp
from jax.experimental import pallas as pl
from jax.experimental.pallas import tpu as pltpu


def _moe_body(x_ref, p_ref, w1_ref, w2_ref, out_ref,
              wg_ref, wu_ref, w2b_ref,
              *, alpha, limit, linear_offset, n_experts):
    e = pl.program_id(0)
    t = pl.program_id(1)
    qcols = w1_ref.shape[2]
    qpairs = qcols // 2
    load_bank = jax.lax.rem(e, 2)
    c = jnp.maximum(e - 1, 0)
    comp_bank = jax.lax.rem(c, 2)

    @pl.when((e < n_experts) & (t < 4))
    def _select_quarter():
        w1q = w1_ref[0].astype(jnp.bfloat16)
        row = jax.lax.broadcasted_iota(jnp.int32, (qcols, qpairs), 0)
        col = jax.lax.broadcasted_iota(jnp.int32, (qcols, qpairs), 1)
        sel_g = (row == 2 * col).astype(jnp.bfloat16)
        sel_u = (row == 2 * col + 1).astype(jnp.bfloat16)
        off = t * qpairs
        wg_ref[load_bank, :, pl.ds(off, qpairs)] = jax.lax.dot_general(
            w1q, sel_g, (((1,), (0,)), ((), ())),
            preferred_element_type=jnp.float32).astype(jnp.bfloat16)
        wu_ref[load_bank, :, pl.ds(off, qpairs)] = jax.lax.dot_general(
            w1q, sel_u, (((1,), (0,)), ((), ())),
            preferred_element_type=jnp.float32).astype(jnp.bfloat16)

    @pl.when((e < n_experts) & (t < 4))
    def _cast_w2_quarter():
        q_rows = w2_ref.shape[1]
        w2b_ref[load_bank, pl.ds(t * q_rows, q_rows), :] = (
            w2_ref[0].astype(jnp.bfloat16))

    @pl.when(e >= 1)
    def _compute_tile():
        x = x_ref[0].astype(jnp.bfloat16)
        gate = jax.lax.dot_general(
            x, wg_ref[comp_bank], (((1,), (0,)), ((), ())),
            preferred_element_type=jnp.float32)
        up = jax.lax.dot_general(
            x, wu_ref[comp_bank], (((1,), (0,)), ((), ())),
            preferred_element_type=jnp.float32)
        gate = jnp.minimum(gate, limit)
        up = jnp.clip(up, -limit, limit)
        glu = gate * jax.nn.sigmoid(alpha * gate)
        inter = glu * (up + linear_offset) * p_ref[0]
        out_ref[0] = jax.lax.dot_general(
            inter.astype(jnp.bfloat16), w2b_ref[comp_bank],
            (((1,), (0,)), ((), ())),
            preferred_element_type=jnp.float32)


def kernel(hidden_states, tokens_per_expert, permuted_probs,
           gate_and_up_projs, down_projs):
    n_experts, dim, two_inter = gate_and_up_projs.shape
    inter = down_projs.shape[1]
    tokens = hidden_states.shape[0]
    tpe = tokens // n_experts
    e_last = n_experts - 1

    bt = 512
    n_tiles = tpe // bt
    x = hidden_states.reshape(n_experts, tpe, dim)
    p = permuted_probs.reshape(n_experts, tpe, 1)

    def _xpo_idx(e, t):
        return (jnp.maximum(e - 1, 0), jnp.where(e == 0, 0, t), 0)

    def _w1_idx(e, t):
        q = jnp.where(e >= n_experts, 3, jnp.clip(t, 0, 3))
        return (jnp.minimum(e, e_last), 0, q)

    def _w2_idx(e, t):
        q = jnp.where(e >= n_experts, 3, jnp.clip(t, 0, 3))
        return (jnp.minimum(e, e_last), q, 0)

    out = pl.pallas_call(
        functools.partial(_moe_body, alpha=1.702, limit=7.0,
                          linear_offset=1.0, n_experts=n_experts),
        grid=(n_experts + 1, n_tiles),
        in_specs=[
            pl.BlockSpec((1, bt, dim), _xpo_idx),
            pl.BlockSpec((1, bt, 1), _xpo_idx),
            pl.BlockSpec((1, dim, two_inter // 4), _w1_idx),
            pl.BlockSpec((1, inter // 4, dim), _w2_idx),
        ],
        out_specs=pl.BlockSpec((1, bt, dim), _xpo_idx),
        out_shape=jax.ShapeDtypeStruct((n_experts, tpe, dim), jnp.float32),
        scratch_shapes=[
            pltpu.VMEM((2, dim, inter), jnp.bfloat16),
            pltpu.VMEM((2, dim, inter), jnp.bfloat16),
            pltpu.VMEM((2, inter, dim), jnp.bfloat16),
        ],
        compiler_params=pltpu.CompilerParams(
            dimension_semantics=("arbitrary", "arbitrary"),
        ),
    )(x, p, gate_and_up_projs, down_projs)
    return out.reshape(tokens, dim)
